# Initial kernel scaffold; baseline (speedup 1.0000x reference)
import jax
import jax.numpy as jnp
from jax.experimental import pallas as pl


def kernel(x, plane1, plane2, plane3):
    raise NotImplementedError("write your pallas kernel here")


# speedup vs baseline: 24.2322x; 24.2322x over previous
import functools

import jax
import jax.numpy as jnp
from jax import lax
from jax.experimental import pallas as pl
from jax.experimental.pallas import tpu as pltpu
from jax.experimental.pallas import tpu_sc as plsc

P = 512
CH = 32
NPTS = 1024 * 1024
NC, NS, L = 2, 16, 16
NW = NC * NS
PTS_PER_W = NPTS // NW
SUP = 1024
NSUP = PTS_PER_W // SUP
SUB = 128
NSUB = SUP // SUB
NG = SUB // L
NTAP = 12


def _coord_terms(g, zero_i, one_i, pm1_i, zero_f):
    ix = ((g + 1.0) * jnp.float32(P) - 1.0) * 0.5
    it = ix.astype(jnp.int32)
    corr = jnp.where(it.astype(jnp.float32) > ix, one_i, zero_i)
    i0 = it - corr
    w = ix - i0.astype(jnp.float32)
    i1 = i0 + one_i
    w0 = jnp.where(i0 >= zero_i, jnp.where(i0 <= pm1_i, 1.0 - w, zero_f), zero_f)
    w1 = jnp.where(i1 >= zero_i, jnp.where(i1 <= pm1_i, w, zero_f), zero_f)
    i0c = jnp.minimum(jnp.maximum(i0, zero_i), pm1_i)
    i1c = jnp.minimum(jnp.maximum(i1, zero_i), pm1_i)
    return i0c, i1c, w0, w1


def _body(xa, xb, xc, p1, p2, p3, out, ca, cb, cc, wv, acc, sem, *tapbufs):
    idx_refs = tapbufs[:NTAP]
    row_refs = tapbufs[NTAP:]
    wid = lax.axis_index("s") * NC + lax.axis_index("c")
    lane = lax.iota(jnp.int32, L)
    zero_i = jnp.full((L,), 0, jnp.int32)
    one_i = jnp.full((L,), 1, jnp.int32)
    pm1_i = jnp.full((L,), P - 1, jnp.int32)
    zero_f = jnp.zeros((L,), jnp.float32)
    planes = (p1, p1, p1, p1, p2, p2, p2, p2, p3, p3, p3, p3)

    def sup_body(sup, carry):
        base = wid * PTS_PER_W + sup * SUP
        pltpu.sync_copy(xa.at[pl.ds(base, SUP)], ca)
        pltpu.sync_copy(xb.at[pl.ds(base, SUP)], cb)
        pltpu.sync_copy(xc.at[pl.ds(base, SUP)], cc)

        def sub_body(sub, carry):
            sbase = sub * SUB

            def idx_body(g, carry):
                off = sbase + g * L
                a0, a1, aw0, aw1 = _coord_terms(
                    ca[pl.ds(off, L)], zero_i, one_i, pm1_i, zero_f)
                b0, b1, bw0, bw1 = _coord_terms(
                    cb[pl.ds(off, L)], zero_i, one_i, pm1_i, zero_f)
                c0, c1, cw0, cw1 = _coord_terms(
                    cc[pl.ds(off, L)], zero_i, one_i, pm1_i, zero_f)
                b0r = b0 * P
                b1r = b1 * P
                c0r = c0 * P
                c1r = c1 * P
                taps = (
                    (b0r + a0, bw0 * aw0), (b0r + a1, bw0 * aw1),
                    (b1r + a0, bw1 * aw0), (b1r + a1, bw1 * aw1),
                    (c0r + a0, cw0 * aw0), (c0r + a1, cw0 * aw1),
                    (c1r + a0, cw1 * aw0), (c1r + a1, cw1 * aw1),
                    (c0r + b0, cw0 * bw0), (c0r + b1, cw0 * bw1),
                    (c1r + b0, cw1 * bw0), (c1r + b1, cw1 * bw1),
                )
                gl = g * L
                for t, (iv, wvv) in enumerate(taps):
                    idx_refs[t][pl.ds(gl, L)] = iv
                    wv[t, pl.ds(gl, L)] = wvv
                return carry

            lax.fori_loop(0, NG, idx_body, 0)

            copies = [
                pltpu.async_copy(planes[t].at[idx_refs[t]], row_refs[t], sem)
                for t in range(NTAP)
            ]
            for cpy in copies:
                cpy.wait()

            def acc_g(g, ridx):
                gl = g * L
                ws = [wv[t, pl.ds(gl, L)] for t in range(NTAP)]

                def acc_ch(chi, cvec):
                    s = ws[0] * plsc.load_gather(row_refs[0], [ridx, cvec])
                    for t in range(1, NTAP):
                        s = s + ws[t] * plsc.load_gather(row_refs[t], [ridx, cvec])
                    acc[chi, pl.ds(sbase + gl, L)] = s
                    return cvec + one_i

                lax.fori_loop(0, CH, acc_ch, zero_i)
                return ridx + L

            lax.fori_loop(0, NG, acc_g, lane)
            return carry

        lax.fori_loop(0, NSUB, sub_body, 0)
        pltpu.sync_copy(acc, out.at[:, pl.ds(base, SUP)])
        return carry

    lax.fori_loop(0, NSUP, sup_body, 0)


@functools.partial(jax.jit, static_argnames=())
def _tri_plane_sample(xa, xb, xc, p1, p2, p3):
    mesh = plsc.VectorSubcoreMesh(
        core_axis_name="c", subcore_axis_name="s",
        num_cores=NC, num_subcores=NS,
    )
    scratch = [
        pltpu.VMEM((SUP,), jnp.float32),
        pltpu.VMEM((SUP,), jnp.float32),
        pltpu.VMEM((SUP,), jnp.float32),
        pltpu.VMEM((NTAP, SUB), jnp.float32),
        pltpu.VMEM((CH, SUP), jnp.float32),
        pltpu.SemaphoreType.DMA,
    ]
    scratch += [pltpu.VMEM((SUB,), jnp.int32) for _ in range(NTAP)]
    scratch += [pltpu.VMEM((SUB, CH), jnp.float32) for _ in range(NTAP)]
    f = pl.kernel(
        _body,
        out_type=jax.ShapeDtypeStruct((CH, NPTS), jnp.float32),
        mesh=mesh,
        scratch_types=scratch,
        compiler_params=pltpu.CompilerParams(
            use_tc_tiling_on_sc=False, needs_layout_passes=False),
    )
    return f(xa, xb, xc, p1, p2, p3)


def kernel(x, plane1, plane2, plane3):
    g = x[0]
    xa = g[..., 0].reshape(-1)
    xb = g[..., 1].reshape(-1)
    xc = g[..., 2].reshape(-1)
    p1 = plane1.reshape(P * P, CH)
    p2 = plane2.reshape(P * P, CH)
    p3 = plane3.reshape(P * P, CH)
    out = _tri_plane_sample(xa, xb, xc, p1, p2, p3)
    return out.reshape(1, CH, 1024, 1024)
